# baseline (speedup 1.0000x reference)
import jax
import jax.numpy as jnp
import numpy as np
from jax.experimental import pallas as pl
from jax.experimental.pallas import tpu as pltpu

N_NODES = 4096
N_EDGES = 131072
EMB = 64
HID = 64
LAYERS = 16
HEADS = 4
K_POOL = 16
D = LAYERS * HID


def _egat_layer(h, ef, src, dst, Wn, Wni, Wnj, We, av, be):
    f_tmp = (h @ Wni)[src] + (h @ Wnj)[dst] + ef @ We + be
    ee = jax.nn.leaky_relu(f_tmp, 0.2)
    logits = jnp.sum(ee * av, axis=-1)
    m = jax.ops.segment_max(logits, dst, num_segments=N_NODES)
    ex = jnp.exp(logits - m[dst])
    s = jax.ops.segment_sum(ex, dst, num_segments=N_NODES)
    alpha = ex / (s[dst] + 1e-9)
    msg = (h @ Wn)[src] * alpha[:, None]
    h_out = jax.ops.segment_sum(msg, dst, num_segments=N_NODES)
    return h_out, f_tmp


def _head_kernel(pooled_ref, Wl_ref, bl_ref, W1_ref, b1_ref, Wc_ref, bc_ref, out_ref):
    z = jnp.maximum(pooled_ref[...] @ Wl_ref[...] + bl_ref[...], 0.0)
    z = jnp.maximum(z @ W1_ref[...] + b1_ref[...], 0.0)
    out_ref[...] = z @ Wc_ref[...] + bc_ref[...]


def kernel(params, h_tokens, e_tokens, edge_index):
    p = params
    src, dst = edge_index[0], edge_index[1]
    h = jax.nn.relu(jnp.take(p['token_emb'], h_tokens, axis=0))
    ef = jnp.take(p['e_token_emb'], e_tokens, axis=0)
    hs = []
    for i in range(LAYERS):
        h, ef = _egat_layer(h, ef, src, dst, p['Wn'][i], p['Wni'][i], p['Wnj'][i],
                            p['We'][i], p['attn'][i], p['be'][i])
        h = jax.nn.relu(h)
        hs.append(h)
    hs = jnp.concatenate(hs, axis=-1)
    dh = D // HEADS
    qk = (hs @ p['Wqk']).reshape(N_NODES, HEADS, dh).transpose(1, 0, 2)
    vv = (hs @ p['Wv']).reshape(N_NODES, HEADS, dh).transpose(1, 0, 2)
    knorm = qk / (jnp.linalg.norm(qk, axis=-1, keepdims=True) + 1e-12)
    dots = jnp.einsum('hid,hjd->hij', qk, knorm) * (dh ** -0.5)
    idx = jnp.arange(N_NODES)
    dots = jnp.where((idx[:, None] < idx[None, :])[None], -1e9, dots)
    dots = jnp.where((idx[:, None] == idx[None, :])[None], -5e4, dots)
    attn = jax.nn.softmax(dots, axis=-1)
    o = jnp.einsum('hij,hjd->hid', attn, vv).transpose(1, 0, 2).reshape(N_NODES, D)
    h2 = o @ p['Wo'] + p['bo']
    t = jax.nn.gelu(h2 @ p['Wi1'] + p['bi1'], approximate=False)
    t = jax.nn.gelu(t @ p['Wi2'] + p['bi2'], approximate=False)
    h3 = jax.nn.relu(t @ p['Wf1'] + p['bf1'])
    fs = jnp.sort(h3, axis=-1)
    _, top_idx = jax.lax.top_k(fs[:, -1], K_POOL)
    pooled = fs[top_idx].reshape(1, K_POOL * HID)

    out = pl.pallas_call(
        _head_kernel,
        out_shape=jax.ShapeDtypeStruct((1, 2), jnp.float32),
    )(pooled, p['Wl'], p['bl'][None, :], p['W1'], p['b1'][None, :],
      p['Wc'], p['bc'][None, :])
    return out


# speedup vs baseline: 1.2745x; 1.2745x over previous
import functools
import jax
import jax.numpy as jnp
from jax import lax
from jax.experimental import pallas as pl
from jax.experimental.pallas import tpu as pltpu
from jax.experimental.pallas import tpu_sc as plsc

N_NODES = 4096
N_EDGES = 131072
EMB = 64
HID = 64
LAYERS = 16
HEADS = 4
K_POOL = 16
D = LAYERS * HID
DH = D // HEADS

NW = 32
CH = 128
EPT = N_EDGES // NW
NCH = EPT // CH

_SC_MESH = plsc.VectorSubcoreMesh(core_axis_name="c", subcore_axis_name="s")


def _transform_body(h_ref, wni_ref, wnj_ref, wn_ref, a_ref, b_ref):
    h = h_ref[...]
    a_ref[...] = jnp.concatenate(
        [jnp.dot(h, wni_ref[...], preferred_element_type=jnp.float32),
         jnp.dot(h, wn_ref[...], preferred_element_type=jnp.float32)], axis=1)
    b_ref[...] = jnp.concatenate(
        [jnp.dot(h, wnj_ref[...], preferred_element_type=jnp.float32),
         jnp.zeros((N_NODES, HID), jnp.float32)], axis=1)


_transform = pl.pallas_call(
    _transform_body,
    out_shape=[jax.ShapeDtypeStruct((N_NODES, 2 * HID), jnp.float32),
               jax.ShapeDtypeStruct((N_NODES, 2 * HID), jnp.float32)],
)


@functools.partial(
    pl.kernel,
    out_type=[jax.ShapeDtypeStruct((N_EDGES, 2 * HID), jnp.float32),
              jax.ShapeDtypeStruct((N_EDGES, 2 * HID), jnp.float32)],
    mesh=_SC_MESH,
    scratch_types=[pltpu.VMEM((CH,), jnp.int32),
                   pltpu.VMEM((CH,), jnp.int32),
                   pltpu.VMEM((CH, 2 * HID), jnp.float32),
                   pltpu.VMEM((CH, 2 * HID), jnp.float32),
                   pltpu.SemaphoreType.DMA,
                   pltpu.SemaphoreType.DMA],
)
def _sc_gather(a_hbm, b_hbm, src_hbm, dst_hbm, oa_hbm, ob_hbm,
               si_v, di_v, a_v, b_v, sem_a, sem_b):
    wid = lax.axis_index("s") * 2 + lax.axis_index("c")
    base = wid * EPT

    def body(j, carry):
        off = base + j * CH
        pltpu.sync_copy(src_hbm.at[pl.ds(off, CH)], si_v)
        pltpu.sync_copy(dst_hbm.at[pl.ds(off, CH)], di_v)
        pltpu.async_copy(a_hbm.at[si_v], a_v, sem_a).wait()
        pltpu.async_copy(b_hbm.at[di_v], b_v, sem_b).wait()
        pltpu.sync_copy(a_v, oa_hbm.at[pl.ds(off, CH)])
        pltpu.sync_copy(b_v, ob_hbm.at[pl.ds(off, CH)])
        return carry

    lax.fori_loop(0, NCH, body, 0)


def _head_body(pooled_ref, wl_ref, bl_ref, w1_ref, b1_ref, wc_ref, bc_ref,
               out_ref):
    z = jnp.maximum(pooled_ref[...] @ wl_ref[...] + bl_ref[...], 0.0)
    z = jnp.maximum(z @ w1_ref[...] + b1_ref[...], 0.0)
    out_ref[...] = z @ wc_ref[...] + bc_ref[...]


def kernel(params, h_tokens, e_tokens, edge_index):
    p = params
    src = edge_index[0]
    dst = edge_index[1]
    h = jax.nn.relu(jnp.take(p['token_emb'], h_tokens, axis=0))
    ef = jnp.take(p['e_token_emb'], e_tokens, axis=0)

    hs_list = []
    for i in range(LAYERS):
        A, B = _transform(h, p['Wni'][i], p['Wnj'][i], p['Wn'][i])
        Ag, Bg = _sc_gather(A, B, src, dst)
        hi = Ag[:, :HID]
        hn = Ag[:, HID:]
        hj = Bg[:, :HID]
        f_tmp = hi + hj + ef @ p['We'][i] + p['be'][i]
        ee = jax.nn.leaky_relu(f_tmp, 0.2)
        logits = jnp.sum(ee * p['attn'][i], axis=-1)
        m = jax.ops.segment_max(logits, dst, num_segments=N_NODES)
        ex = jnp.exp(logits - m[dst])
        s = jax.ops.segment_sum(ex, dst, num_segments=N_NODES)
        alpha = ex / (s[dst] + 1e-9)
        msg = hn * alpha[:, None]
        h_out = jax.ops.segment_sum(msg, dst, num_segments=N_NODES)
        h = jax.nn.relu(h_out)
        ef = f_tmp
        hs_list.append(h)
    hs = jnp.concatenate(hs_list, axis=-1)

    qk = (hs @ p['Wqk']).reshape(N_NODES, HEADS, DH).transpose(1, 0, 2)
    vv = (hs @ p['Wv']).reshape(N_NODES, HEADS, DH).transpose(1, 0, 2)
    knorm = qk / (jnp.linalg.norm(qk, axis=-1, keepdims=True) + 1e-12)
    dots = jnp.einsum('hid,hjd->hij', qk, knorm) * (DH ** -0.5)
    idx = jnp.arange(N_NODES)
    dots = jnp.where((idx[:, None] < idx[None, :])[None], -1e9, dots)
    dots = jnp.where((idx[:, None] == idx[None, :])[None], -5e4, dots)
    attn = jax.nn.softmax(dots, axis=-1)
    o = jnp.einsum('hij,hjd->hid', attn, vv).transpose(1, 0, 2).reshape(N_NODES, D)
    h2 = o @ p['Wo'] + p['bo']
    t = jax.nn.gelu(h2 @ p['Wi1'] + p['bi1'], approximate=False)
    t = jax.nn.gelu(t @ p['Wi2'] + p['bi2'], approximate=False)
    h3 = jax.nn.relu(t @ p['Wf1'] + p['bf1'])
    fs = jnp.sort(h3, axis=-1)
    _, top_idx = jax.lax.top_k(fs[:, -1], K_POOL)
    pooled = fs[top_idx].reshape(1, K_POOL * HID)

    out = pl.pallas_call(
        _head_body,
        out_shape=jax.ShapeDtypeStruct((1, 2), jnp.float32),
    )(pooled, p['Wl'], p['bl'][None, :], p['W1'], p['b1'][None, :],
      p['Wc'], p['bc'][None, :])
    return out
